# baseline (speedup 1.0000x reference)
import functools

import jax
import jax.numpy as jnp
from jax import lax
from jax.experimental import pallas as pl
from jax.experimental.pallas import tpu as pltpu
from jax.experimental.pallas import tpu_sc as plsc

N_PAPER = 10000
N_AUTHOR = 10000
E = 160000
D = 256
HALF = 128

NC = 2
NS = 16
L = 16

PAD_N = 10240
ROWS_PER_SUB = PAD_N // NS
CHUNK = 80
EDGES_PER_SUB = E // NS
NCHUNK = EDGES_PER_SUB // CHUNK


NBUF = 3
NGROUP = NCHUNK // NBUF


def _sc_body(xa_ref, xp_ref, esw_ref, edw_ref, esc_ref, edc_ref, sums_ref, cnts_ref,
             acc, sidx_all, didxs, rowss, sems, sem2s, sem3s):
    c = lax.axis_index("c")
    s = lax.axis_index("s")
    row0 = s * ROWS_PER_SUB
    ebase = s * EDGES_PER_SUB
    rows0 = rowss[0]

    def _fill_rows(rows, val):
        def _f(i, carry):
            r = i // (HALF // L)
            k = i % (HALF // L)
            rows[r, pl.ds(k * L, L)] = jnp.full((L,), val, jnp.float32)
            return carry
        lax.fori_loop(0, CHUNK * (HALF // L), _f, 0)

    def _zero_acc():
        _fill_rows(rows0, 0.0)
        descs = [pltpu.async_copy(
            rows0, acc.at[pl.ds(row0 + i * CHUNK, CHUNK), :], sems[i % NBUF])
            for i in range(ROWS_PER_SUB // CHUNK)]
        for d in descs:
            d.wait()

    def _copy_out(dst_ref):
        descs = [pltpu.async_copy(
            acc.at[pl.ds(row0 + i * 128, 128), :],
            dst_ref.at[pl.ds(row0 + i * 128, 128), :], sems[i % NBUF])
            for i in range(ROWS_PER_SUB // 128)]
        for d in descs:
            d.wait()

    col0 = pl.multiple_of(c * HALF, HALF)
    for rel, (x_ref, es_ref, ed_ref) in enumerate(
            ((xa_ref, esw_ref, edw_ref), (xp_ref, esc_ref, edc_ref))):
        _zero_acc()
        plsc.subcore_barrier()

        pltpu.sync_copy(es_ref.at[pl.ds(ebase, EDGES_PER_SUB)], sidx_all)

        def _group(g, carry, nbuf):
            ddescs, descs = [], []
            for u in range(nbuf):
                j = g * NBUF + u
                ddescs.append(pltpu.async_copy(
                    ed_ref.at[pl.ds(ebase + j * CHUNK, CHUNK)],
                    didxs[u], sem3s[u]))
                descs.append(pltpu.async_copy(
                    x_ref.at[sidx_all.at[pl.ds(j * CHUNK, CHUNK)],
                             pl.ds(col0, HALF)],
                    rowss[u], sems[u]))
            sdescs = []
            for u in range(nbuf):
                ddescs[u].wait()
                descs[u].wait()
                sdescs.append(pltpu.async_copy(
                    rowss[u], acc.at[didxs[u]], sem2s[u], add=True))
            for d in sdescs:
                d.wait()
            return carry
        lax.fori_loop(0, NGROUP, functools.partial(_group, nbuf=NBUF), 0)
        _group(NGROUP, 0, NCHUNK - NGROUP * NBUF)

        plsc.subcore_barrier()
        _copy_out(sums_ref.at[rel, c])

    _zero_acc()
    plsc.subcore_barrier()
    _fill_rows(rows0, 1.0)

    def _cnt_chunks(ed_ref):
        def _group(g, carry, nbuf):
            ddescs = []
            for u in range(nbuf):
                base = ebase + (g * NBUF + u) * CHUNK
                ddescs.append(pltpu.async_copy(
                    ed_ref.at[pl.ds(base, CHUNK)], didxs[u], sem3s[u]))
            descs = []
            for u in range(nbuf):
                ddescs[u].wait()
                descs.append(
                    pltpu.async_copy(rows0, acc.at[didxs[u]], sems[u],
                                     add=True))
            for d in descs:
                d.wait()
            return carry
        lax.fori_loop(0, NGROUP, functools.partial(_group, nbuf=NBUF), 0)
        _group(NGROUP, 0, NCHUNK - NGROUP * NBUF)

    @pl.when(c == 0)
    def _():
        _cnt_chunks(edw_ref)

    @pl.when(c == 1)
    def _():
        _cnt_chunks(edc_ref)

    plsc.subcore_barrier()
    _copy_out(cnts_ref.at[c])


_sc_aggregate = pl.kernel(
    _sc_body,
    out_type=(
        jax.ShapeDtypeStruct((2, NC, PAD_N, HALF), jnp.float32),
        jax.ShapeDtypeStruct((NC, PAD_N, HALF), jnp.float32),
    ),
    mesh=plsc.VectorSubcoreMesh(
        core_axis_name="c", subcore_axis_name="s",
        num_cores=NC, num_subcores=NS),
    scratch_types=[
        pltpu.VMEM_SHARED((PAD_N, HALF), jnp.float32),
        pltpu.VMEM((EDGES_PER_SUB,), jnp.int32),
        [pltpu.VMEM((CHUNK,), jnp.int32)] * NBUF,
        [pltpu.VMEM((CHUNK, HALF), jnp.float32)] * NBUF,
        [pltpu.SemaphoreType.DMA] * NBUF,
        [pltpu.SemaphoreType.DMA] * NBUF,
        [pltpu.SemaphoreType.DMA] * NBUF,
    ],
)


_BM = 2000
_DN = (((1,), (1,)), ((), ()))
_MM = functools.partial(lax.dot_general, dimension_numbers=_DN,
                        preferred_element_type=jnp.float32,
                        precision=lax.Precision.HIGHEST)


def _tc_root_body(xp_ref, xa_ref, wp_ref, bp_ref, wa_ref, ba_ref,
                  opr_ref, oa_ref):
    opr_ref[...] = _MM(xp_ref[...], wp_ref[...]) + bp_ref[...]
    oa_ref[...] = _MM(xa_ref[...], wa_ref[...]) + ba_ref[...]


def _tc_root(xp, xa, Wp, bp, Wa, ba):
    row_spec = pl.BlockSpec((_BM, D), lambda i: (i, 0))
    w_spec = pl.BlockSpec((D, D), lambda i: (0, 0))
    b_spec = pl.BlockSpec((1, D), lambda i: (0, 0))
    return pl.pallas_call(
        _tc_root_body,
        grid=(N_PAPER // _BM,),
        in_specs=[row_spec, row_spec, w_spec, b_spec, w_spec, b_spec],
        out_specs=[row_spec, row_spec],
        out_shape=[
            jax.ShapeDtypeStruct((N_PAPER, D), jnp.float32),
            jax.ShapeDtypeStruct((N_AUTHOR, D), jnp.float32),
        ],
    )(xp, xa, Wp, bp.reshape(1, D), Wa, ba.reshape(1, D))


def _tc_rel_body(opr_ref, swl_ref, swh_ref, scl_ref, sch_ref,
                 cw_ref, cc_ref, wwl_ref, wwh_ref, wcl_ref, wch_ref,
                 op_ref):
    inv_w = 1.0 / jnp.maximum(cw_ref[0][:, 0:1], 1.0)
    inv_c = 1.0 / jnp.maximum(cc_ref[0][:, 0:1], 1.0)
    out = opr_ref[...]
    out += _MM(swl_ref[0, 0] * inv_w, wwl_ref[...])
    out += _MM(swh_ref[0, 0] * inv_w, wwh_ref[...])
    out += _MM(scl_ref[0, 0] * inv_c, wcl_ref[...])
    out += _MM(sch_ref[0, 0] * inv_c, wch_ref[...])
    op_ref[...] = out


def _tc_rel(opr, sums, cnts, Ww, Wc):
    BM = _BM
    row_spec = pl.BlockSpec((BM, D), lambda i: (i, 0))
    wh_spec = lambda h: pl.BlockSpec((D, HALF), lambda i: (0, h))
    sum_spec = lambda r, h: pl.BlockSpec((1, 1, BM, HALF),
                                         lambda i: (r, h, i, 0))
    cnt_spec = lambda r: pl.BlockSpec((1, BM, HALF), lambda i: (r, i, 0))
    return pl.pallas_call(
        _tc_rel_body,
        grid=(N_PAPER // BM,),
        in_specs=[row_spec,
                  sum_spec(0, 0), sum_spec(0, 1),
                  sum_spec(1, 0), sum_spec(1, 1),
                  cnt_spec(0), cnt_spec(1),
                  wh_spec(0), wh_spec(1), wh_spec(0), wh_spec(1)],
        out_specs=row_spec,
        out_shape=jax.ShapeDtypeStruct((N_PAPER, D), jnp.float32),
    )(opr, sums, sums, sums, sums, cnts, cnts, Ww, Ww, Wc, Wc)


def kernel(x_paper, x_author, edge_index_writes, edge_index_cites,
           W_root_paper, b_root_paper, W_root_author, b_root_author,
           W_rel_writes, W_rel_cites):
    eiw = edge_index_writes.astype(jnp.int32)
    eic = edge_index_cites.astype(jnp.int32)
    sums, cnts = _sc_aggregate(x_author, x_paper,
                               eiw[0], eiw[1], eic[0], eic[1])
    opr, out_a = _tc_root(x_paper, x_author, W_root_paper, b_root_paper,
                          W_root_author, b_root_author)
    out_p = _tc_rel(opr, sums, cnts, W_rel_writes, W_rel_cites)
    return out_p, out_a


# speedup vs baseline: 1.0439x; 1.0439x over previous
import functools

import jax
import jax.numpy as jnp
from jax import lax
from jax.experimental import pallas as pl
from jax.experimental.pallas import tpu as pltpu
from jax.experimental.pallas import tpu_sc as plsc

N_PAPER = 10000
N_AUTHOR = 10000
E = 160000
D = 256
HALF = 128

NC = 2
NS = 16
L = 16

PAD_N = 10240
ROWS_PER_SUB = PAD_N // NS
CHUNK = 40
EDGES_PER_SUB = E // NS
NCHUNK = EDGES_PER_SUB // CHUNK


NBUF = 6
NGROUP = NCHUNK // NBUF


def _sc_body(xa_ref, xp_ref, esw_ref, edw_ref, esc_ref, edc_ref, sums_ref, cnts_ref,
             acc, sidx_all, didxs, rowss, sems, sem2s, sem3s):
    c = lax.axis_index("c")
    s = lax.axis_index("s")
    row0 = s * ROWS_PER_SUB
    ebase = s * EDGES_PER_SUB
    rows0 = rowss[0]

    def _fill_rows(rows, val):
        def _f(i, carry):
            r = i // (HALF // L)
            k = i % (HALF // L)
            rows[r, pl.ds(k * L, L)] = jnp.full((L,), val, jnp.float32)
            return carry
        lax.fori_loop(0, CHUNK * (HALF // L), _f, 0)

    def _zero_acc():
        _fill_rows(rows0, 0.0)

        def _z(i, carry):
            pltpu.sync_copy(rows0, acc.at[pl.ds(row0 + i * CHUNK, CHUNK), :])
            return carry
        lax.fori_loop(0, ROWS_PER_SUB // CHUNK, _z, 0)

    def _copy_out(dst_ref):
        def _o(i, carry):
            sl = pl.ds(row0 + i * 128, 128)
            pltpu.sync_copy(acc.at[sl, :], dst_ref.at[sl, :])
            return carry
        lax.fori_loop(0, ROWS_PER_SUB // 128, _o, 0)

    col0 = pl.multiple_of(c * HALF, HALF)
    for rel, (x_ref, es_ref, ed_ref) in enumerate(
            ((xa_ref, esw_ref, edw_ref), (xp_ref, esc_ref, edc_ref))):
        _zero_acc()
        plsc.subcore_barrier()

        pltpu.sync_copy(es_ref.at[pl.ds(ebase, EDGES_PER_SUB)], sidx_all)

        def _group(g, carry, nbuf):
            ddescs, descs = [], []
            for u in range(nbuf):
                j = g * NBUF + u
                ddescs.append(pltpu.async_copy(
                    ed_ref.at[pl.ds(ebase + j * CHUNK, CHUNK)],
                    didxs[u], sem3s[u]))
                descs.append(pltpu.async_copy(
                    x_ref.at[sidx_all.at[pl.ds(j * CHUNK, CHUNK)],
                             pl.ds(col0, HALF)],
                    rowss[u], sems[u]))
            sdescs = []
            for u in range(nbuf):
                ddescs[u].wait()
                descs[u].wait()
                sdescs.append(pltpu.async_copy(
                    rowss[u], acc.at[didxs[u]], sem2s[u], add=True))
            for d in sdescs:
                d.wait()
            return carry
        lax.fori_loop(0, NGROUP, functools.partial(_group, nbuf=NBUF), 0)
        _group(NGROUP, 0, NCHUNK - NGROUP * NBUF)

        plsc.subcore_barrier()
        _copy_out(sums_ref.at[rel, c])

    _zero_acc()
    plsc.subcore_barrier()
    _fill_rows(rows0, 1.0)

    def _cnt_chunks(ed_ref):
        def _group(g, carry, nbuf):
            ddescs = []
            for u in range(nbuf):
                base = ebase + (g * NBUF + u) * CHUNK
                ddescs.append(pltpu.async_copy(
                    ed_ref.at[pl.ds(base, CHUNK)], didxs[u], sem3s[u]))
            descs = []
            for u in range(nbuf):
                ddescs[u].wait()
                descs.append(
                    pltpu.async_copy(rows0, acc.at[didxs[u]], sems[u],
                                     add=True))
            for d in descs:
                d.wait()
            return carry
        lax.fori_loop(0, NGROUP, functools.partial(_group, nbuf=NBUF), 0)
        _group(NGROUP, 0, NCHUNK - NGROUP * NBUF)

    @pl.when(c == 0)
    def _():
        _cnt_chunks(edw_ref)

    @pl.when(c == 1)
    def _():
        _cnt_chunks(edc_ref)

    plsc.subcore_barrier()
    _copy_out(cnts_ref.at[c])


_sc_aggregate = pl.kernel(
    _sc_body,
    out_type=(
        jax.ShapeDtypeStruct((2, NC, PAD_N, HALF), jnp.float32),
        jax.ShapeDtypeStruct((NC, PAD_N, HALF), jnp.float32),
    ),
    mesh=plsc.VectorSubcoreMesh(
        core_axis_name="c", subcore_axis_name="s",
        num_cores=NC, num_subcores=NS),
    scratch_types=[
        pltpu.VMEM_SHARED((PAD_N, HALF), jnp.float32),
        pltpu.VMEM((EDGES_PER_SUB,), jnp.int32),
        [pltpu.VMEM((CHUNK,), jnp.int32)] * NBUF,
        [pltpu.VMEM((CHUNK, HALF), jnp.float32)] * NBUF,
        [pltpu.SemaphoreType.DMA] * NBUF,
        [pltpu.SemaphoreType.DMA] * NBUF,
        [pltpu.SemaphoreType.DMA] * NBUF,
    ],
)


_BM = 2000
_DN = (((1,), (1,)), ((), ()))
_MM = functools.partial(lax.dot_general, dimension_numbers=_DN,
                        preferred_element_type=jnp.float32,
                        precision=lax.Precision.HIGHEST)


def _tc_root_body(xp_ref, xa_ref, wp_ref, bp_ref, wa_ref, ba_ref,
                  opr_ref, oa_ref):
    opr_ref[...] = _MM(xp_ref[...], wp_ref[...]) + bp_ref[...]
    oa_ref[...] = _MM(xa_ref[...], wa_ref[...]) + ba_ref[...]


def _tc_root(xp, xa, Wp, bp, Wa, ba):
    row_spec = pl.BlockSpec((_BM, D), lambda i: (i, 0))
    w_spec = pl.BlockSpec((D, D), lambda i: (0, 0))
    b_spec = pl.BlockSpec((1, D), lambda i: (0, 0))
    return pl.pallas_call(
        _tc_root_body,
        grid=(N_PAPER // _BM,),
        in_specs=[row_spec, row_spec, w_spec, b_spec, w_spec, b_spec],
        out_specs=[row_spec, row_spec],
        out_shape=[
            jax.ShapeDtypeStruct((N_PAPER, D), jnp.float32),
            jax.ShapeDtypeStruct((N_AUTHOR, D), jnp.float32),
        ],
    )(xp, xa, Wp, bp.reshape(1, D), Wa, ba.reshape(1, D))


def _tc_rel_body(opr_ref, swl_ref, swh_ref, scl_ref, sch_ref,
                 cw_ref, cc_ref, wwl_ref, wwh_ref, wcl_ref, wch_ref,
                 op_ref):
    inv_w = 1.0 / jnp.maximum(cw_ref[0][:, 0:1], 1.0)
    inv_c = 1.0 / jnp.maximum(cc_ref[0][:, 0:1], 1.0)
    out = opr_ref[...]
    out += _MM(swl_ref[0, 0] * inv_w, wwl_ref[...])
    out += _MM(swh_ref[0, 0] * inv_w, wwh_ref[...])
    out += _MM(scl_ref[0, 0] * inv_c, wcl_ref[...])
    out += _MM(sch_ref[0, 0] * inv_c, wch_ref[...])
    op_ref[...] = out


def _tc_rel(opr, sums, cnts, Ww, Wc):
    BM = _BM
    row_spec = pl.BlockSpec((BM, D), lambda i: (i, 0))
    wh_spec = lambda h: pl.BlockSpec((D, HALF), lambda i: (0, h))
    sum_spec = lambda r, h: pl.BlockSpec((1, 1, BM, HALF),
                                         lambda i: (r, h, i, 0))
    cnt_spec = lambda r: pl.BlockSpec((1, BM, HALF), lambda i: (r, i, 0))
    return pl.pallas_call(
        _tc_rel_body,
        grid=(N_PAPER // BM,),
        in_specs=[row_spec,
                  sum_spec(0, 0), sum_spec(0, 1),
                  sum_spec(1, 0), sum_spec(1, 1),
                  cnt_spec(0), cnt_spec(1),
                  wh_spec(0), wh_spec(1), wh_spec(0), wh_spec(1)],
        out_specs=row_spec,
        out_shape=jax.ShapeDtypeStruct((N_PAPER, D), jnp.float32),
    )(opr, sums, sums, sums, sums, cnts, cnts, Ww, Ww, Wc, Wc)


def kernel(x_paper, x_author, edge_index_writes, edge_index_cites,
           W_root_paper, b_root_paper, W_root_author, b_root_author,
           W_rel_writes, W_rel_cites):
    eiw = edge_index_writes.astype(jnp.int32)
    eic = edge_index_cites.astype(jnp.int32)
    sums, cnts = _sc_aggregate(x_author, x_paper,
                               eiw[0], eiw[1], eic[0], eic[1])
    opr, out_a = _tc_root(x_paper, x_author, W_root_paper, b_root_paper,
                          W_root_author, b_root_author)
    out_p = _tc_rel(opr, sums, cnts, W_rel_writes, W_rel_cites)
    return out_p, out_a


# speedup vs baseline: 1.0807x; 1.0353x over previous
import functools

import jax
import jax.numpy as jnp
from jax import lax
from jax.experimental import pallas as pl
from jax.experimental.pallas import tpu as pltpu
from jax.experimental.pallas import tpu_sc as plsc

N_PAPER = 10000
N_AUTHOR = 10000
E = 160000
D = 256
HALF = 128

NC = 2
NS = 16
L = 16

PAD_N = 10240
ROWS_PER_SUB = PAD_N // NS
CHUNK = 40
EDGES_PER_SUB = E // NS
NCHUNK = EDGES_PER_SUB // CHUNK


NBUF = 7
NGROUP = NCHUNK // NBUF


def _sc_body(xa_ref, xp_ref, esw_ref, edw_ref, esc_ref, edc_ref, sums_ref, cnts_ref,
             acc, sidx_all, didxs, rowss, sems, sem2s, sem3s):
    c = lax.axis_index("c")
    s = lax.axis_index("s")
    row0 = s * ROWS_PER_SUB
    ebase = s * EDGES_PER_SUB
    rows0 = rowss[0]

    def _fill_rows(rows, val):
        def _f(i, carry):
            r = i // (HALF // L)
            k = i % (HALF // L)
            rows[r, pl.ds(k * L, L)] = jnp.full((L,), val, jnp.float32)
            return carry
        lax.fori_loop(0, CHUNK * (HALF // L), _f, 0)

    def _zero_acc():
        _fill_rows(rows0, 0.0)

        def _z(i, carry):
            pltpu.sync_copy(rows0, acc.at[pl.ds(row0 + i * CHUNK, CHUNK), :])
            return carry
        lax.fori_loop(0, ROWS_PER_SUB // CHUNK, _z, 0)

    def _copy_out(dst_ref):
        def _o(i, carry):
            sl = pl.ds(row0 + i * 128, 128)
            pltpu.sync_copy(acc.at[sl, :], dst_ref.at[sl, :])
            return carry
        lax.fori_loop(0, ROWS_PER_SUB // 128, _o, 0)

    col0 = pl.multiple_of(c * HALF, HALF)
    for rel, (x_ref, es_ref, ed_ref) in enumerate(
            ((xa_ref, esw_ref, edw_ref), (xp_ref, esc_ref, edc_ref))):
        _zero_acc()
        plsc.subcore_barrier()

        pltpu.sync_copy(es_ref.at[pl.ds(ebase, EDGES_PER_SUB)], sidx_all)

        def _group(g, carry, nbuf):
            ddescs, descs = [], []
            for u in range(nbuf):
                j = g * NBUF + u
                ddescs.append(pltpu.async_copy(
                    ed_ref.at[pl.ds(ebase + j * CHUNK, CHUNK)],
                    didxs[u], sem3s[u]))
                descs.append(pltpu.async_copy(
                    x_ref.at[sidx_all.at[pl.ds(j * CHUNK, CHUNK)],
                             pl.ds(col0, HALF)],
                    rowss[u], sems[u]))
            sdescs = []
            for u in range(nbuf):
                ddescs[u].wait()
                descs[u].wait()
                sdescs.append(pltpu.async_copy(
                    rowss[u], acc.at[didxs[u]], sem2s[u], add=True))
            for d in sdescs:
                d.wait()
            return carry
        lax.fori_loop(0, NGROUP, functools.partial(_group, nbuf=NBUF), 0)
        _group(NGROUP, 0, NCHUNK - NGROUP * NBUF)

        plsc.subcore_barrier()
        _copy_out(sums_ref.at[rel, c])

    _zero_acc()
    plsc.subcore_barrier()
    _fill_rows(rows0, 1.0)

    def _cnt_chunks(ed_ref):
        def _group(g, carry, nbuf):
            ddescs = []
            for u in range(nbuf):
                base = ebase + (g * NBUF + u) * CHUNK
                ddescs.append(pltpu.async_copy(
                    ed_ref.at[pl.ds(base, CHUNK)], didxs[u], sem3s[u]))
            descs = []
            for u in range(nbuf):
                ddescs[u].wait()
                descs.append(
                    pltpu.async_copy(rows0, acc.at[didxs[u]], sems[u],
                                     add=True))
            for d in descs:
                d.wait()
            return carry
        lax.fori_loop(0, NGROUP, functools.partial(_group, nbuf=NBUF), 0)
        _group(NGROUP, 0, NCHUNK - NGROUP * NBUF)

    @pl.when(c == 0)
    def _():
        _cnt_chunks(edw_ref)

    @pl.when(c == 1)
    def _():
        _cnt_chunks(edc_ref)

    plsc.subcore_barrier()
    _copy_out(cnts_ref.at[c])


_sc_aggregate = pl.kernel(
    _sc_body,
    out_type=(
        jax.ShapeDtypeStruct((2, NC, PAD_N, HALF), jnp.float32),
        jax.ShapeDtypeStruct((NC, PAD_N, HALF), jnp.float32),
    ),
    mesh=plsc.VectorSubcoreMesh(
        core_axis_name="c", subcore_axis_name="s",
        num_cores=NC, num_subcores=NS),
    scratch_types=[
        pltpu.VMEM_SHARED((PAD_N, HALF), jnp.float32),
        pltpu.VMEM((EDGES_PER_SUB,), jnp.int32),
        [pltpu.VMEM((CHUNK,), jnp.int32)] * NBUF,
        [pltpu.VMEM((CHUNK, HALF), jnp.float32)] * NBUF,
        [pltpu.SemaphoreType.DMA] * NBUF,
        [pltpu.SemaphoreType.DMA] * NBUF,
        [pltpu.SemaphoreType.DMA] * NBUF,
    ],
)


_BM = 2000
_DN = (((1,), (1,)), ((), ()))
_MM = functools.partial(lax.dot_general, dimension_numbers=_DN,
                        preferred_element_type=jnp.float32,
                        precision=lax.Precision.HIGHEST)


def _tc_root_body(xp_ref, xa_ref, wp_ref, bp_ref, wa_ref, ba_ref,
                  opr_ref, oa_ref):
    opr_ref[...] = _MM(xp_ref[...], wp_ref[...]) + bp_ref[...]
    oa_ref[...] = _MM(xa_ref[...], wa_ref[...]) + ba_ref[...]


def _tc_root(xp, xa, Wp, bp, Wa, ba):
    row_spec = pl.BlockSpec((_BM, D), lambda i: (i, 0))
    w_spec = pl.BlockSpec((D, D), lambda i: (0, 0))
    b_spec = pl.BlockSpec((1, D), lambda i: (0, 0))
    return pl.pallas_call(
        _tc_root_body,
        grid=(N_PAPER // _BM,),
        in_specs=[row_spec, row_spec, w_spec, b_spec, w_spec, b_spec],
        out_specs=[row_spec, row_spec],
        out_shape=[
            jax.ShapeDtypeStruct((N_PAPER, D), jnp.float32),
            jax.ShapeDtypeStruct((N_AUTHOR, D), jnp.float32),
        ],
    )(xp, xa, Wp, bp.reshape(1, D), Wa, ba.reshape(1, D))


def _tc_rel_body(opr_ref, swl_ref, swh_ref, scl_ref, sch_ref,
                 cw_ref, cc_ref, wwl_ref, wwh_ref, wcl_ref, wch_ref,
                 op_ref):
    inv_w = 1.0 / jnp.maximum(cw_ref[0][:, 0:1], 1.0)
    inv_c = 1.0 / jnp.maximum(cc_ref[0][:, 0:1], 1.0)
    out = opr_ref[...]
    out += _MM(swl_ref[0, 0] * inv_w, wwl_ref[...])
    out += _MM(swh_ref[0, 0] * inv_w, wwh_ref[...])
    out += _MM(scl_ref[0, 0] * inv_c, wcl_ref[...])
    out += _MM(sch_ref[0, 0] * inv_c, wch_ref[...])
    op_ref[...] = out


def _tc_rel(opr, sums, cnts, Ww, Wc):
    BM = _BM
    row_spec = pl.BlockSpec((BM, D), lambda i: (i, 0))
    wh_spec = lambda h: pl.BlockSpec((D, HALF), lambda i: (0, h))
    sum_spec = lambda r, h: pl.BlockSpec((1, 1, BM, HALF),
                                         lambda i: (r, h, i, 0))
    cnt_spec = lambda r: pl.BlockSpec((1, BM, HALF), lambda i: (r, i, 0))
    return pl.pallas_call(
        _tc_rel_body,
        grid=(N_PAPER // BM,),
        in_specs=[row_spec,
                  sum_spec(0, 0), sum_spec(0, 1),
                  sum_spec(1, 0), sum_spec(1, 1),
                  cnt_spec(0), cnt_spec(1),
                  wh_spec(0), wh_spec(1), wh_spec(0), wh_spec(1)],
        out_specs=row_spec,
        out_shape=jax.ShapeDtypeStruct((N_PAPER, D), jnp.float32),
    )(opr, sums, sums, sums, sums, cnts, cnts, Ww, Ww, Wc, Wc)


def kernel(x_paper, x_author, edge_index_writes, edge_index_cites,
           W_root_paper, b_root_paper, W_root_author, b_root_author,
           W_rel_writes, W_rel_cites):
    eiw = edge_index_writes.astype(jnp.int32)
    eic = edge_index_cites.astype(jnp.int32)
    sums, cnts = _sc_aggregate(x_author, x_paper,
                               eiw[0], eiw[1], eic[0], eic[1])
    opr, out_a = _tc_root(x_paper, x_author, W_root_paper, b_root_paper,
                          W_root_author, b_root_author)
    out_p = _tc_rel(opr, sums, cnts, W_rel_writes, W_rel_cites)
    return out_p, out_a
